# baseline (speedup 1.0000x reference)
import jax
import jax.numpy as jnp
from jax import lax
from jax.experimental import pallas as pl
from jax.experimental.pallas import tpu as pltpu
from jax.experimental.pallas import tpu_sc as plsc

B = 16384
K = 32
NC, NS, L = 2, 16, 16
NW = NC * NS
CHUNK = B // NW
IDX_W = 128
IDX_ROWS = CHUNK // IDX_W
GROUPS = CHUNK // L
GLOBAL_MEAN = 3.5


def _mf_body(uid_hbm, iid_hbm, w_hbm, u_hbm, bu_hbm, bi_hbm, out_hbm,
             uid_v, iid_v, wrows_v, urows_v, bu_v, bi_v, out_v, sem):
    wid = lax.axis_index("s") * NC + lax.axis_index("c")
    row0 = wid * IDX_ROWS
    pltpu.sync_copy(uid_hbm.at[pl.ds(row0, IDX_ROWS)], uid_v)
    pltpu.sync_copy(iid_hbm.at[pl.ds(row0, IDX_ROWS)], iid_v)

    copies = []
    for j in range(IDX_ROWS):
        sl = pl.ds(j * IDX_W, IDX_W)
        copies.append(pltpu.async_copy(w_hbm.at[uid_v.at[j]], wrows_v.at[sl], sem))
        copies.append(pltpu.async_copy(u_hbm.at[iid_v.at[j]], urows_v.at[sl], sem))
        copies.append(pltpu.async_copy(bu_hbm.at[uid_v.at[j]], bu_v.at[sl], sem))
        copies.append(pltpu.async_copy(bi_hbm.at[iid_v.at[j]], bi_v.at[sl], sem))
    for c in copies:
        c.wait()

    lane = lax.iota(jnp.int32, L)

    def group(g, carry):
        o = g * L
        rid = o + lane
        acc = bu_v[pl.ds(o, L)] + bi_v[pl.ds(o, L)] + jnp.float32(GLOBAL_MEAN)
        for k in range(K):
            kv = jnp.full((L,), k, jnp.int32)
            acc = acc + (plsc.load_gather(wrows_v, [rid, kv]) *
                         plsc.load_gather(urows_v, [rid, kv]))
        out_v[pl.ds(o, L)] = acc
        return carry

    lax.fori_loop(0, GROUPS, group, 0)
    pltpu.sync_copy(out_v, out_hbm.at[pl.ds(wid * CHUNK, CHUNK)])


def kernel(user_ids, item_ids, W, U, bias_user, bias_item):
    uid2 = user_ids.astype(jnp.int32).reshape(NW * IDX_ROWS, IDX_W)
    iid2 = item_ids.astype(jnp.int32).reshape(NW * IDX_ROWS, IDX_W)
    mesh = plsc.VectorSubcoreMesh(core_axis_name="c", subcore_axis_name="s",
                                  num_cores=NC, num_subcores=NS)
    f = pl.kernel(
        _mf_body,
        out_type=jax.ShapeDtypeStruct((B,), jnp.float32),
        mesh=mesh,
        compiler_params=pltpu.CompilerParams(needs_layout_passes=False,
                                             use_tc_tiling_on_sc=False),
        scratch_types=[
            pltpu.VMEM((IDX_ROWS, IDX_W), jnp.int32),
            pltpu.VMEM((IDX_ROWS, IDX_W), jnp.int32),
            pltpu.VMEM((CHUNK, K), jnp.float32),
            pltpu.VMEM((CHUNK, K), jnp.float32),
            pltpu.VMEM((CHUNK,), jnp.float32),
            pltpu.VMEM((CHUNK,), jnp.float32),
            pltpu.VMEM((CHUNK,), jnp.float32),
            pltpu.SemaphoreType.DMA,
        ],
    )
    return f(uid2, iid2, W, U, bias_user, bias_item)
